# Initial kernel scaffold; baseline (speedup 1.0000x reference)
import jax
import jax.numpy as jnp
from jax.experimental import pallas as pl


def kernel(x_user, x_claim, params, e0_uc, e0_cu, e1_uc, e1_cu):
    raise NotImplementedError("write your pallas kernel here")


# speedup vs baseline: 4.2857x; 4.2857x over previous
import functools
import math

import jax
import jax.numpy as jnp
from jax import lax
from jax.experimental import pallas as pl
from jax.experimental.pallas import tpu as pltpu
from jax.experimental.pallas import tpu_sc as plsc

_N = 10000
_DEG = 16
_D = 128
_H = 256

_NC = 2
_NS = 16
_NW = _NC * _NS
_CHUNK = 128


def _sc_gather(table, idx):
    _, Dd = table.shape
    B = idx.shape[0]
    assert B % _CHUNK == 0
    n_chunks = B // _CHUNK
    npw = -(-n_chunks // _NW)
    mesh = plsc.VectorSubcoreMesh(core_axis_name="c", subcore_axis_name="s")

    @functools.partial(
        pl.kernel,
        mesh=mesh,
        out_type=jax.ShapeDtypeStruct((B, Dd), jnp.float32),
        scratch_types=[
            pltpu.VMEM((_CHUNK,), jnp.int32),
            pltpu.VMEM((_CHUNK, Dd), jnp.float32),
            pltpu.SemaphoreType.DMA,
        ],
    )
    def gather_k(table_hbm, idx_hbm, out_hbm, idx_v, rows_v, sem):
        wid = lax.axis_index("s") * _NC + lax.axis_index("c")

        def body(j, carry):
            cid = j * _NW + wid

            @pl.when(cid < n_chunks)
            def _do():
                base = cid * _CHUNK
                pltpu.sync_copy(idx_hbm.at[pl.ds(base, _CHUNK)], idx_v)
                pltpu.async_copy(table_hbm.at[idx_v], rows_v, sem).wait()
                pltpu.sync_copy(rows_v, out_hbm.at[pl.ds(base, _CHUNK)])

            return carry

        lax.fori_loop(0, npw, body, 0)

    return gather_k(table, idx)


_SQRT1_2 = 1.0 / math.sqrt(2.0)


def _gelu(y):
    return 0.5 * y * (1.0 + lax.erf(y * _SQRT1_2))


def _sage_layer(msgs_t, x_dst, wcat, blstm, wout, bout, ln_g, ln_b, want_stats):
    T, N, d = msgs_t.shape
    H = wout.shape[0]
    BLK = 200
    grid = N // BLK

    def body(msgs_ref, xdst_ref, wcat_ref, blstm_ref, wout_ref, bout_ref,
             lng_ref, lnb_ref, out_ref, *stat_refs):
        h = jnp.zeros((BLK, d), jnp.float32)
        c = jnp.zeros((BLK, d), jnp.float32)
        wc = wcat_ref[...]
        bl = blstm_ref[...]
        for t in range(T):
            cat = jnp.concatenate([msgs_ref[t], h], axis=1)
            g = lax.dot_general(cat, wc, (((1,), (1,)), ((), ())),
                                preferred_element_type=jnp.float32) + bl
            i = jax.nn.sigmoid(g[:, :d])
            f = jax.nn.sigmoid(g[:, d:2 * d])
            gg = jnp.tanh(g[:, 2 * d:3 * d])
            o = jax.nn.sigmoid(g[:, 3 * d:])
            c = f * c + i * gg
            h = o * jnp.tanh(c)
        cat2 = jnp.concatenate([xdst_ref[...], h], axis=1)
        y = lax.dot_general(cat2, wout_ref[...], (((1,), (1,)), ((), ())),
                            preferred_element_type=jnp.float32) + bout_ref[...]
        y = _gelu(y)
        m = jnp.mean(y, axis=-1, keepdims=True)
        v = jnp.mean((y - m) ** 2, axis=-1, keepdims=True)
        y = (y - m) * lax.rsqrt(v + 1e-5) * lng_ref[...] + lnb_ref[...]
        out_ref[...] = y
        if want_stats:
            sum_ref, sq_ref = stat_refs

            @pl.when(pl.program_id(0) == 0)
            def _init():
                sum_ref[...] = jnp.zeros_like(sum_ref)
                sq_ref[...] = jnp.zeros_like(sq_ref)

            sum_ref[...] += jnp.sum(y, axis=0, keepdims=True)
            sq_ref[...] += jnp.sum(y * y, axis=0, keepdims=True)

    out_shape = [jax.ShapeDtypeStruct((N, H), jnp.float32)]
    out_specs = [pl.BlockSpec((BLK, H), lambda i: (i, 0))]
    if want_stats:
        out_shape += [jax.ShapeDtypeStruct((1, H), jnp.float32)] * 2
        out_specs += [pl.BlockSpec((1, H), lambda i: (0, 0)),
                      pl.BlockSpec((1, H), lambda i: (0, 0))]

    return pl.pallas_call(
        body,
        grid=(grid,),
        in_specs=[
            pl.BlockSpec((T, BLK, d), lambda i: (0, i, 0)),
            pl.BlockSpec((BLK, d), lambda i: (i, 0)),
            pl.BlockSpec((4 * d, 2 * d), lambda i: (0, 0)),
            pl.BlockSpec((1, 4 * d), lambda i: (0, 0)),
            pl.BlockSpec((H, 2 * d), lambda i: (0, 0)),
            pl.BlockSpec((1, H), lambda i: (0, 0)),
            pl.BlockSpec((1, H), lambda i: (0, 0)),
            pl.BlockSpec((1, H), lambda i: (0, 0)),
        ],
        out_shape=out_shape,
        out_specs=out_specs,
        compiler_params=pltpu.CompilerParams(
            dimension_semantics=("arbitrary",)),
    )(msgs_t, x_dst, wcat, blstm, wout, bout, ln_g, ln_b)


def _head(x, ssum, ssq, w1, b1, w2, b2, bn_g, bn_b):
    N, H = x.shape
    BLK = 200
    grid = N // BLK
    inv_n = 1.0 / N

    def body(x_ref, s_ref, q_ref, w1_ref, b1_ref, w2_ref, b2_ref,
             g_ref, bb_ref, out_ref):
        m = s_ref[...] * inv_n
        v = q_ref[...] * inv_n - m * m
        xx = (x_ref[...] - m) * lax.rsqrt(v + 1e-5) * g_ref[...] + bb_ref[...]
        y = lax.dot_general(xx, w1_ref[...], (((1,), (1,)), ((), ())),
                            preferred_element_type=jnp.float32) + b1_ref[...]
        y = _gelu(y)
        z = lax.dot_general(y, w2_ref[...], (((1,), (1,)), ((), ())),
                            preferred_element_type=jnp.float32) + b2_ref[0, 0]
        out_ref[...] = z

    return pl.pallas_call(
        body,
        grid=(grid,),
        in_specs=[
            pl.BlockSpec((BLK, H), lambda i: (i, 0)),
            pl.BlockSpec((1, H), lambda i: (0, 0)),
            pl.BlockSpec((1, H), lambda i: (0, 0)),
            pl.BlockSpec((H, H), lambda i: (0, 0)),
            pl.BlockSpec((1, H), lambda i: (0, 0)),
            pl.BlockSpec((8, H), lambda i: (0, 0)),
            pl.BlockSpec(memory_space=pltpu.SMEM),
            pl.BlockSpec((1, H), lambda i: (0, 0)),
            pl.BlockSpec((1, H), lambda i: (0, 0)),
        ],
        out_shape=jax.ShapeDtypeStruct((N, 8), jnp.float32),
        out_specs=pl.BlockSpec((BLK, 8), lambda i: (i, 0)),
    )(x, ssum, ssq, w1, b1, w2, b2, bn_g, bn_b)


def kernel(x_user, x_claim, params, e0_uc, e0_cu, e1_uc, e1_cu):
    p = params

    def prep(pre, din):
        wcat = jnp.concatenate([p[pre + '_Wih'], p[pre + '_Whh']], axis=1)
        blstm = p[pre + '_blstm'].reshape(1, 4 * din)
        wout = jnp.concatenate([p[pre + '_Wself'], p[pre + '_Wneigh']], axis=1)
        bout = p[pre + '_b'].reshape(1, _H)
        return wcat, blstm, wout, bout

    ln_g = p['ln_g'].reshape(1, _H)
    ln_b = p['ln_b'].reshape(1, _H)

    i0u = e0_uc.T.reshape(-1).astype(jnp.int32)
    i0c = e0_cu.T.reshape(-1).astype(jnp.int32)
    i1u = e1_uc.T.reshape(-1).astype(jnp.int32)

    g0u = _sc_gather(x_user, i0u).reshape(_DEG, _N, _D)
    g0c = _sc_gather(x_claim, i0c).reshape(_DEG, _N, _D)

    wcat, blstm, wout, bout = prep('l1_uc', _D)
    (h_claim,) = _sage_layer(g0u, x_claim, wcat, blstm, wout, bout,
                             ln_g, ln_b, False)
    wcat, blstm, wout, bout = prep('l1_cu', _D)
    (h_user,) = _sage_layer(g0c, x_user, wcat, blstm, wout, bout,
                            ln_g, ln_b, False)

    g1 = _sc_gather(h_user, i1u).reshape(_DEG, _N, _H)
    wcat, blstm, wout, bout = prep('l2_uc', _H)
    h2, ssum, ssq = _sage_layer(g1, h_claim, wcat, blstm, wout, bout,
                                ln_g, ln_b, True)

    w2p = jnp.pad(p['fc2_W'], ((0, 7), (0, 0)))
    z = _head(h2, ssum, ssq, p['fc1_W'], p['fc1_b'].reshape(1, _H),
              w2p, p['fc2_b'].reshape(1, 1),
              p['bn_g'].reshape(1, _H), p['bn_b'].reshape(1, _H))
    return z[:, :1]
